# baseline (speedup 1.0000x reference)
import functools

import jax
import jax.numpy as jnp
from jax import lax
from jax.experimental import pallas as pl
from jax.experimental.pallas import tpu as pltpu
from jax.experimental.pallas import tpu_sc as plsc

_NT = 4
_N = 10000
_E = 80000
_D = 128
_LV = 3
_CH = 128
_CPT = 40
_NC = 2
_NS = 16
_EP = _NS * _CPT * _CH
_RPT = 624
_PAD_N = 10112
_ZR = _PAD_N // _NS
_DW = 16

_f32 = jnp.float32


_ROWS_BLK = 2000


def _dot(a, b):
    return lax.dot_general(a, b, (((1,), (0,)), ((), ())),
                           preferred_element_type=_f32)


def _embed_body(x_ref, w_ref, b_ref, o_ref):
    o_ref[...] = jnp.tanh(_dot(x_ref[...], w_ref[...]) + b_ref[...])


def _embed(x, w, b):
    grid = (_N // _ROWS_BLK,)
    return pl.pallas_call(
        _embed_body,
        grid=grid,
        in_specs=[
            pl.BlockSpec((_ROWS_BLK, _D), lambda i: (i, 0)),
            pl.BlockSpec((_D, _D), lambda i: (0, 0)),
            pl.BlockSpec((1, _D), lambda i: (0, 0)),
        ],
        out_specs=pl.BlockSpec((_ROWS_BLK, _D), lambda i: (i, 0)),
        out_shape=jax.ShapeDtypeStruct((_N, _D), _f32),
    )(x, w, b)


def _merge_body(g0, g1, g2, g3, d0, d1, d2, d3, h_ref,
                wc_ref, bc_ref, wm_ref, bm_ref, o_ref):
    acc = h_ref[...] + bm_ref[...]
    for t, (g, dg) in enumerate(((g0, d0), (g1, d1), (g2, d2), (g3, d3))):
        m = _dot(g[...], wc_ref[:, t * _D:(t + 1) * _D]) \
            + dg[:, 0:1] * bc_ref[:, t * _D:(t + 1) * _D]
        acc = acc + _dot(jnp.tanh(m), wm_ref[t * _D:(t + 1) * _D, :])
    o_ref[...] = jnp.tanh(acc)


def _merge(gs, degs, h, wc, bc, wm, bm):
    grid = (_N // _ROWS_BLK,)
    return pl.pallas_call(
        _merge_body,
        grid=grid,
        in_specs=[pl.BlockSpec((_ROWS_BLK, _D), lambda i: (i, 0))] * _NT
        + [pl.BlockSpec((_ROWS_BLK, _DW), lambda i: (i, 0))] * _NT + [
            pl.BlockSpec((_ROWS_BLK, _D), lambda i: (i, 0)),
            pl.BlockSpec((_D, _NT * _D), lambda i: (0, 0)),
            pl.BlockSpec((1, _NT * _D), lambda i: (0, 0)),
            pl.BlockSpec((_NT * _D, _D), lambda i: (0, 0)),
            pl.BlockSpec((1, _D), lambda i: (0, 0)),
        ],
        out_specs=pl.BlockSpec((_ROWS_BLK, _D), lambda i: (i, 0)),
        out_shape=jax.ShapeDtypeStruct((_N, _D), _f32),
    )(*gs, *degs, h, wc, bc, wm, bm)


def _spmm_body(h_hbm, srcp, dstp, zrow,
               out0, out1, out2, out3,
               src1d_v, dst2d_v, rows_v, acc_sh, sem):
    c = lax.axis_index("c")
    s = lax.axis_index("s")
    outs = (out0, out1, out2, out3)

    nj = jnp.minimum(s * 0 + _CPT, _CPT)

    def _issue(j, p):
        i0 = pl.multiple_of(j * _CH, 8)
        b0 = pl.multiple_of(p * _CH, 8)
        pltpu.make_async_copy(
            h_hbm.at[src1d_v.at[pl.ds(i0, _CH)]],
            rows_v.at[pl.ds(b0, _CH)], sem).start()

    def _wait_scatter(j, p):
        i0 = pl.multiple_of(j * _CH, 8)
        b0 = pl.multiple_of(p * _CH, 8)
        pltpu.make_async_copy(
            h_hbm.at[src1d_v.at[pl.ds(i0, _CH)]],
            rows_v.at[pl.ds(b0, _CH)], sem).wait()
        pltpu.sync_copy(rows_v.at[pl.ds(b0, _CH)],
                        acc_sh.at[dst2d_v.at[j]], add=True)

    for t in range(_NT):
        @pl.when(c == t // 2)
        def _process():
            e0 = pl.multiple_of((t * _NS + s) * _CPT * _CH, 8)
            pltpu.sync_copy(srcp.at[pl.ds(e0, _CPT * _CH)], src1d_v)
            r0 = t * (_EP // _CH) + s * _CPT
            pltpu.sync_copy(dstp.at[pl.ds(r0, _CPT)], dst2d_v)

            pltpu.sync_copy(zrow, rows_v.at[pl.ds(0, _CH)])
            for m in range(5):
                rn = _CH if m < 4 else _ZR - 4 * _CH
                pltpu.sync_copy(rows_v.at[pl.ds(0, rn)],
                                acc_sh.at[pl.ds(s * _ZR + m * _CH, rn)])
            plsc.subcore_barrier()

            _issue(0, 0)

            def body(j, carry):
                p = j & 1

                @pl.when(j + 1 < _CPT)
                def _prefetch():
                    _issue(j + 1, 1 - p)

                _wait_scatter(j, p)
                return carry

            lax.fori_loop(0, nj, body, 0)
            plsc.subcore_barrier()

            for m in range(5):
                r0e = s * _RPT + m * _CH
                rn = _CH if m < 4 else _RPT - 4 * _CH
                pltpu.sync_copy(acc_sh.at[pl.ds(r0e, rn)],
                                rows_v.at[pl.ds(0, rn)])
                pltpu.sync_copy(rows_v.at[pl.ds(0, rn)],
                                outs[t].at[pl.ds(r0e, rn)])

            @pl.when(s == 0)
            def _tail():
                rt = _N - _NS * _RPT
                pltpu.sync_copy(acc_sh.at[pl.ds(_NS * _RPT, rt)],
                                rows_v.at[pl.ds(0, rt)])
                pltpu.sync_copy(rows_v.at[pl.ds(0, rt)],
                                outs[t].at[pl.ds(_NS * _RPT, rt)])

            plsc.subcore_barrier()


@functools.lru_cache(maxsize=1)
def _spmm_call():
    return pl.kernel(
        _spmm_body,
        out_type=[jax.ShapeDtypeStruct((_N, _D), _f32)] * _NT,
        mesh=plsc.VectorSubcoreMesh(core_axis_name="c", subcore_axis_name="s",
                                    num_cores=_NC, num_subcores=_NS),
        scratch_types=[
            pltpu.VMEM((_CPT * _CH,), jnp.int32),
            pltpu.VMEM((_CPT, _CH), jnp.int32),
            pltpu.VMEM((2 * _CH, _D), _f32),
            pltpu.VMEM_SHARED((_PAD_N, _D), _f32),
            pltpu.SemaphoreType.DMA,
        ],
    )


def _deg_body(dstp, zrow16, ones16,
              deg0, deg1, deg2, deg3,
              dst2d_v, ones_v, dstage_v, dacc_sh):
    c = lax.axis_index("c")
    s = lax.axis_index("s")
    degs = (deg0, deg1, deg2, deg3)

    pltpu.sync_copy(ones16, ones_v)
    pltpu.sync_copy(zrow16, dstage_v)

    for t in range(_NT):
        @pl.when(c == t // 2)
        def _process():
            r0 = t * (_EP // _CH) + s * _CPT
            pltpu.sync_copy(dstp.at[pl.ds(r0, _CPT)], dst2d_v)

            for m in range(5):
                rn = _CH if m < 4 else _ZR - 4 * _CH
                pltpu.sync_copy(dstage_v.at[pl.ds(0, rn)],
                                dacc_sh.at[pl.ds(s * _ZR + m * _CH, rn)])
            plsc.subcore_barrier()

            def body(j, carry):
                pltpu.sync_copy(ones_v, dacc_sh.at[dst2d_v.at[j]], add=True)
                return carry

            lax.fori_loop(0, _CPT, body, 0)
            plsc.subcore_barrier()

            for m in range(5):
                r0e = s * _RPT + m * _CH
                rn = _CH if m < 4 else _RPT - 4 * _CH
                pltpu.sync_copy(dacc_sh.at[pl.ds(r0e, rn)],
                                dstage_v.at[pl.ds(0, rn)])
                pltpu.sync_copy(dstage_v.at[pl.ds(0, rn)],
                                degs[t].at[pl.ds(r0e, rn)])

            @pl.when(s == 0)
            def _tail():
                rt = _N - _NS * _RPT
                pltpu.sync_copy(dacc_sh.at[pl.ds(_NS * _RPT, rt)],
                                dstage_v.at[pl.ds(0, rt)])
                pltpu.sync_copy(dstage_v.at[pl.ds(0, rt)],
                                degs[t].at[pl.ds(_NS * _RPT, rt)])

            pltpu.sync_copy(zrow16, dstage_v)
            plsc.subcore_barrier()


@functools.lru_cache(maxsize=1)
def _deg_call():
    return pl.kernel(
        _deg_body,
        out_type=[jax.ShapeDtypeStruct((_N, _DW), _f32)] * _NT,
        mesh=plsc.VectorSubcoreMesh(core_axis_name="c", subcore_axis_name="s",
                                    num_cores=_NC, num_subcores=_NS),
        scratch_types=[
            pltpu.VMEM((_CPT, _CH), jnp.int32),
            pltpu.VMEM((_CH, _DW), _f32),
            pltpu.VMEM((_CH, _DW), _f32),
            pltpu.VMEM_SHARED((_PAD_N, _DW), _f32),
        ],
    )


def kernel(node_feat, edge_index, w_n2l_W, w_n2l_b, conv_W, conv_b,
           merge_W, merge_b):
    pad = _EP - _E
    srcp = jnp.concatenate(
        [edge_index[:, 0, :],
         jnp.zeros((_NT, pad), jnp.int32)], axis=1).reshape(-1)
    pad_rows = _N + jnp.arange(pad, dtype=jnp.int32) % (_PAD_N - _N)
    dstp = jnp.concatenate(
        [edge_index[:, 1, :],
         jnp.broadcast_to(pad_rows, (_NT, pad))], axis=1).reshape(-1, _CH)
    zrow = jnp.zeros((_CH, _D), _f32)
    zrow16 = jnp.zeros((_CH, _DW), _f32)
    ones16 = jnp.ones((_CH, _DW), _f32)

    h = _embed(node_feat, w_n2l_W, w_n2l_b.reshape(1, _D))
    degs = _deg_call()(dstp, zrow16, ones16)
    for lv in range(_LV):
        gs = _spmm_call()(h, srcp, dstp, zrow)
        h = _merge(gs, degs, h, conv_W[lv], conv_b[lv].reshape(1, _NT * _D),
                   merge_W[lv], merge_b[lv].reshape(1, _D))
    return h


# speedup vs baseline: 2.6354x; 2.6354x over previous
import functools

import jax
import jax.numpy as jnp
from jax import lax
from jax.experimental import pallas as pl
from jax.experimental.pallas import tpu as pltpu
from jax.experimental.pallas import tpu_sc as plsc

_NT = 4
_N = 10000
_E = 80000
_D = 128
_LV = 3
_CH = 128
_NCH = _E // _CH
_CPT = 40
_NC = 2
_NS = 16
_EP = _NS * _CPT * _CH
_RPT = 624
_PAD_N = 10112
_ZR = _PAD_N // _NS
_DW = 16

_f32 = jnp.float32


_ROWS_BLK = 2000


def _dot(a, b):
    return lax.dot_general(a, b, (((1,), (0,)), ((), ())),
                           preferred_element_type=_f32)


def _embed_body(x_ref, w_ref, b_ref, o_ref):
    o_ref[...] = jnp.tanh(_dot(x_ref[...], w_ref[...]) + b_ref[...])


def _embed(x, w, b):
    grid = (_N // _ROWS_BLK,)
    return pl.pallas_call(
        _embed_body,
        grid=grid,
        in_specs=[
            pl.BlockSpec((_ROWS_BLK, _D), lambda i: (i, 0)),
            pl.BlockSpec((_D, _D), lambda i: (0, 0)),
            pl.BlockSpec((1, _D), lambda i: (0, 0)),
        ],
        out_specs=pl.BlockSpec((_ROWS_BLK, _D), lambda i: (i, 0)),
        out_shape=jax.ShapeDtypeStruct((_N, _D), _f32),
    )(x, w, b)


def _merge_body(g0, g1, g2, g3, d0, d1, d2, d3, h_ref,
                wc_ref, bc_ref, wm_ref, bm_ref, o_ref):
    acc = h_ref[...] + bm_ref[...]
    for t, (g, dg) in enumerate(((g0, d0), (g1, d1), (g2, d2), (g3, d3))):
        m = _dot(g[...], wc_ref[:, t * _D:(t + 1) * _D]) \
            + dg[:, 0:1] * bc_ref[:, t * _D:(t + 1) * _D]
        acc = acc + _dot(jnp.tanh(m), wm_ref[t * _D:(t + 1) * _D, :])
    o_ref[...] = jnp.tanh(acc)


def _merge(gs, degs, h, wc, bc, wm, bm):
    grid = (_N // _ROWS_BLK,)
    return pl.pallas_call(
        _merge_body,
        grid=grid,
        in_specs=[pl.BlockSpec((_ROWS_BLK, _D), lambda i: (i, 0))] * _NT
        + [pl.BlockSpec((_ROWS_BLK, _DW), lambda i: (i, 0))] * _NT + [
            pl.BlockSpec((_ROWS_BLK, _D), lambda i: (i, 0)),
            pl.BlockSpec((_D, _NT * _D), lambda i: (0, 0)),
            pl.BlockSpec((1, _NT * _D), lambda i: (0, 0)),
            pl.BlockSpec((_NT * _D, _D), lambda i: (0, 0)),
            pl.BlockSpec((1, _D), lambda i: (0, 0)),
        ],
        out_specs=pl.BlockSpec((_ROWS_BLK, _D), lambda i: (i, 0)),
        out_shape=jax.ShapeDtypeStruct((_N, _D), _f32),
    )(*gs, *degs, h, wc, bc, wm, bm)


def _spmm_body(h_hbm, src, dst, zrow,
               out0, out1, out2, out3,
               idx_v, dst2d_v, rows_v, acc_sh, sem, sem2):
    c = lax.axis_index("c")
    s = lax.axis_index("s")
    outs = (out0, out1, out2, out3)

    nj = jnp.where(s < _NCH - 16 * (_NCH // 16), _NCH // 16 + 1, _NCH // 16)

    def _idx_slices(t, j):
        ch = s + j * _NS
        e0 = pl.multiple_of(t * _E + ch * _CH, 8)
        q = j & 3
        b0 = pl.multiple_of(q * _CH, 8)
        return (pltpu.make_async_copy(src.at[pl.ds(e0, _CH)],
                                      idx_v.at[pl.ds(b0, _CH)], sem2),
                pltpu.make_async_copy(dst.at[pl.ds(e0, _CH)],
                                      dst2d_v.at[q], sem2))

    def _gather(j, p):
        q = j & 3
        b0 = pl.multiple_of(q * _CH, 8)
        r0 = pl.multiple_of(p * _CH, 8)
        return pltpu.make_async_copy(
            h_hbm.at[idx_v.at[pl.ds(b0, _CH)]],
            rows_v.at[pl.ds(r0, _CH)], sem)

    for t in range(_NT):
        @pl.when(c == t // 2)
        def _process():
            pltpu.sync_copy(zrow, rows_v.at[pl.ds(0, _CH)])
            for m in range(5):
                rn = _CH if m < 4 else _ZR - 4 * _CH
                pltpu.sync_copy(rows_v.at[pl.ds(0, rn)],
                                acc_sh.at[pl.ds(s * _ZR + m * _CH, rn)])
            plsc.subcore_barrier()

            ca, cb = _idx_slices(t, 0)
            ca.start(); cb.start(); ca.wait(); cb.wait()
            _gather(0, 0).start()

            @pl.when(nj > 1)
            def _pre():
                na, nb = _idx_slices(t, 1)
                na.start(); nb.start()

            def body(j, carry):
                p = j & 1

                @pl.when(j + 2 < nj)
                def _prefetch_idx():
                    na, nb = _idx_slices(t, j + 2)
                    na.start(); nb.start()

                @pl.when(j + 1 < nj)
                def _launch_next():
                    na, nb = _idx_slices(t, j + 1)
                    na.wait(); nb.wait()
                    _gather(j + 1, 1 - p).start()

                _gather(j, p).wait()
                q = j & 3
                b0 = pl.multiple_of(p * _CH, 8)
                pltpu.sync_copy(rows_v.at[pl.ds(b0, _CH)],
                                acc_sh.at[dst2d_v.at[q]], add=True)
                return carry

            lax.fori_loop(0, nj, body, 0)
            plsc.subcore_barrier()

            for m in range(5):
                r0e = s * _RPT + m * _CH
                rn = _CH if m < 4 else _RPT - 4 * _CH
                pltpu.sync_copy(acc_sh.at[pl.ds(r0e, rn)],
                                rows_v.at[pl.ds(0, rn)])
                pltpu.sync_copy(rows_v.at[pl.ds(0, rn)],
                                outs[t].at[pl.ds(r0e, rn)])

            @pl.when(s == 0)
            def _tail():
                rt = _N - _NS * _RPT
                pltpu.sync_copy(acc_sh.at[pl.ds(_NS * _RPT, rt)],
                                rows_v.at[pl.ds(0, rt)])
                pltpu.sync_copy(rows_v.at[pl.ds(0, rt)],
                                outs[t].at[pl.ds(_NS * _RPT, rt)])

            plsc.subcore_barrier()


@functools.lru_cache(maxsize=1)
def _spmm_call():
    return pl.kernel(
        _spmm_body,
        out_type=[jax.ShapeDtypeStruct((_N, _D), _f32)] * _NT,
        mesh=plsc.VectorSubcoreMesh(core_axis_name="c", subcore_axis_name="s",
                                    num_cores=_NC, num_subcores=_NS),
        scratch_types=[
            pltpu.VMEM((4 * _CH,), jnp.int32),
            pltpu.VMEM((4, _CH), jnp.int32),
            pltpu.VMEM((2 * _CH, _D), _f32),
            pltpu.VMEM_SHARED((_PAD_N, _D), _f32),
            pltpu.SemaphoreType.DMA,
            pltpu.SemaphoreType.DMA,
        ],
    )


def _deg_body(dstp, zrow16, ones16,
              deg0, deg1, deg2, deg3,
              dst2d_v, ones_v, dstage_v, dacc_sh):
    c = lax.axis_index("c")
    s = lax.axis_index("s")
    degs = (deg0, deg1, deg2, deg3)

    pltpu.sync_copy(ones16, ones_v)
    pltpu.sync_copy(zrow16, dstage_v)

    for t in range(_NT):
        @pl.when(c == t // 2)
        def _process():
            r0 = t * (_EP // _CH) + s * _CPT
            pltpu.sync_copy(dstp.at[pl.ds(r0, _CPT)], dst2d_v)

            for m in range(5):
                rn = _CH if m < 4 else _ZR - 4 * _CH
                pltpu.sync_copy(dstage_v.at[pl.ds(0, rn)],
                                dacc_sh.at[pl.ds(s * _ZR + m * _CH, rn)])
            plsc.subcore_barrier()

            def body(j, carry):
                pltpu.sync_copy(ones_v, dacc_sh.at[dst2d_v.at[j]], add=True)
                return carry

            lax.fori_loop(0, _CPT, body, 0)
            plsc.subcore_barrier()

            for m in range(5):
                r0e = s * _RPT + m * _CH
                rn = _CH if m < 4 else _RPT - 4 * _CH
                pltpu.sync_copy(dacc_sh.at[pl.ds(r0e, rn)],
                                dstage_v.at[pl.ds(0, rn)])
                pltpu.sync_copy(dstage_v.at[pl.ds(0, rn)],
                                degs[t].at[pl.ds(r0e, rn)])

            @pl.when(s == 0)
            def _tail():
                rt = _N - _NS * _RPT
                pltpu.sync_copy(dacc_sh.at[pl.ds(_NS * _RPT, rt)],
                                dstage_v.at[pl.ds(0, rt)])
                pltpu.sync_copy(dstage_v.at[pl.ds(0, rt)],
                                degs[t].at[pl.ds(_NS * _RPT, rt)])

            pltpu.sync_copy(zrow16, dstage_v)
            plsc.subcore_barrier()


@functools.lru_cache(maxsize=1)
def _deg_call():
    return pl.kernel(
        _deg_body,
        out_type=[jax.ShapeDtypeStruct((_N, _DW), _f32)] * _NT,
        mesh=plsc.VectorSubcoreMesh(core_axis_name="c", subcore_axis_name="s",
                                    num_cores=_NC, num_subcores=_NS),
        scratch_types=[
            pltpu.VMEM((_CPT, _CH), jnp.int32),
            pltpu.VMEM((_CH, _DW), _f32),
            pltpu.VMEM((_CH, _DW), _f32),
            pltpu.VMEM_SHARED((_PAD_N, _DW), _f32),
        ],
    )


def kernel(node_feat, edge_index, w_n2l_W, w_n2l_b, conv_W, conv_b,
           merge_W, merge_b):
    src = edge_index[:, 0, :].reshape(-1)
    dst = edge_index[:, 1, :].reshape(-1)
    pad = _EP - _E
    pad_rows = _N + jnp.arange(pad, dtype=jnp.int32) % (_PAD_N - _N)
    dstp = jnp.concatenate(
        [edge_index[:, 1, :],
         jnp.broadcast_to(pad_rows, (_NT, pad))], axis=1).reshape(-1, _CH)
    zrow = jnp.zeros((_CH, _D), _f32)
    zrow16 = jnp.zeros((_CH, _DW), _f32)
    ones16 = jnp.ones((_CH, _DW), _f32)

    h = _embed(node_feat, w_n2l_W, w_n2l_b.reshape(1, _D))
    degs = _deg_call()(dstp, zrow16, ones16)
    for lv in range(_LV):
        gs = _spmm_call()(h, src, dst, zrow)
        h = _merge(gs, degs, h, conv_W[lv], conv_b[lv].reshape(1, _NT * _D),
                   merge_W[lv], merge_b[lv].reshape(1, _D))
    return h


# speedup vs baseline: 2.6360x; 1.0003x over previous
import functools

import jax
import jax.numpy as jnp
from jax import lax
from jax.experimental import pallas as pl
from jax.experimental.pallas import tpu as pltpu
from jax.experimental.pallas import tpu_sc as plsc

_NT = 4
_N = 10000
_E = 80000
_D = 128
_LV = 3
_CH = 128
_NCH = _E // _CH
_CPT = 40
_NC = 2
_NS = 16
_EP = _NS * _CPT * _CH
_RPT = 624
_PAD_N = 10112
_ZR = _PAD_N // _NS
_DW = 16

_f32 = jnp.float32


_ROWS_BLK = 2000


def _dot(a, b):
    return lax.dot_general(a, b, (((1,), (0,)), ((), ())),
                           preferred_element_type=_f32)


def _embed_body(x_ref, w_ref, b_ref, o_ref):
    o_ref[...] = jnp.tanh(_dot(x_ref[...], w_ref[...]) + b_ref[...])


def _embed(x, w, b):
    grid = (_N // _ROWS_BLK,)
    return pl.pallas_call(
        _embed_body,
        grid=grid,
        in_specs=[
            pl.BlockSpec((_ROWS_BLK, _D), lambda i: (i, 0)),
            pl.BlockSpec((_D, _D), lambda i: (0, 0)),
            pl.BlockSpec((1, _D), lambda i: (0, 0)),
        ],
        out_specs=pl.BlockSpec((_ROWS_BLK, _D), lambda i: (i, 0)),
        out_shape=jax.ShapeDtypeStruct((_N, _D), _f32),
    )(x, w, b)


def _merge_body(g0, g1, g2, g3, d0, d1, d2, d3, h_ref,
                wc_ref, bc_ref, wm_ref, bm_ref, o_ref):
    acc = h_ref[...] + bm_ref[...]
    for t, (g, dg) in enumerate(((g0, d0), (g1, d1), (g2, d2), (g3, d3))):
        m = _dot(g[...], wc_ref[:, t * _D:(t + 1) * _D]) \
            + dg[:, 0:1] * bc_ref[:, t * _D:(t + 1) * _D]
        acc = acc + _dot(jnp.tanh(m), wm_ref[t * _D:(t + 1) * _D, :])
    o_ref[...] = jnp.tanh(acc)


def _merge(gs, degs, h, wc, bc, wm, bm):
    grid = (_N // _ROWS_BLK,)
    return pl.pallas_call(
        _merge_body,
        grid=grid,
        in_specs=[pl.BlockSpec((_ROWS_BLK, _D), lambda i: (i, 0))] * _NT
        + [pl.BlockSpec((_ROWS_BLK, _DW), lambda i: (i, 0))] * _NT + [
            pl.BlockSpec((_ROWS_BLK, _D), lambda i: (i, 0)),
            pl.BlockSpec((_D, _NT * _D), lambda i: (0, 0)),
            pl.BlockSpec((1, _NT * _D), lambda i: (0, 0)),
            pl.BlockSpec((_NT * _D, _D), lambda i: (0, 0)),
            pl.BlockSpec((1, _D), lambda i: (0, 0)),
        ],
        out_specs=pl.BlockSpec((_ROWS_BLK, _D), lambda i: (i, 0)),
        out_shape=jax.ShapeDtypeStruct((_N, _D), _f32),
    )(*gs, *degs, h, wc, bc, wm, bm)


def _spmm_body(h_hbm, src, dst, zrow,
               out0, out1, out2, out3,
               idx_v, dst2d_v, rows_v, acc_sh, sem, sem2):
    c = lax.axis_index("c")
    s = lax.axis_index("s")
    outs = (out0, out1, out2, out3)

    nj = jnp.where(s < _NCH - 16 * (_NCH // 16), _NCH // 16 + 1, _NCH // 16)

    def _idx_slices(t, j):
        ch = s + j * _NS
        e0 = pl.multiple_of(t * _E + ch * _CH, 8)
        q = j & 3
        b0 = pl.multiple_of(q * _CH, 8)
        return (pltpu.make_async_copy(src.at[pl.ds(e0, _CH)],
                                      idx_v.at[pl.ds(b0, _CH)], sem2),
                pltpu.make_async_copy(dst.at[pl.ds(e0, _CH)],
                                      dst2d_v.at[q], sem2))

    def _gather(j, p):
        q = j & 3
        b0 = pl.multiple_of(q * _CH, 8)
        r0 = pl.multiple_of(p * _CH, 8)
        return pltpu.make_async_copy(
            h_hbm.at[idx_v.at[pl.ds(b0, _CH)]],
            rows_v.at[pl.ds(r0, _CH)], sem)

    for t in range(_NT):
        @pl.when(c == t // 2)
        def _process():
            pltpu.sync_copy(zrow, rows_v.at[pl.ds(0, _CH)])
            for m in range(5):
                rn = _CH if m < 4 else _ZR - 4 * _CH
                pltpu.sync_copy(rows_v.at[pl.ds(0, rn)],
                                acc_sh.at[pl.ds(s * _ZR + m * _CH, rn)])
            plsc.subcore_barrier()

            ca, cb = _idx_slices(t, 0)
            ca.start(); cb.start(); ca.wait(); cb.wait()
            _gather(0, 0).start()

            @pl.when(nj > 1)
            def _pre():
                na, nb = _idx_slices(t, 1)
                na.start(); nb.start()

            def body(j, carry):
                p = j & 1

                @pl.when(j + 2 < nj)
                def _prefetch_idx():
                    na, nb = _idx_slices(t, j + 2)
                    na.start(); nb.start()

                @pl.when(j + 1 < nj)
                def _launch_next():
                    na, nb = _idx_slices(t, j + 1)
                    na.wait(); nb.wait()
                    _gather(j + 1, 1 - p).start()

                _gather(j, p).wait()
                q = j & 3
                b0 = pl.multiple_of(p * _CH, 8)
                pltpu.sync_copy(rows_v.at[pl.ds(b0, _CH)],
                                acc_sh.at[dst2d_v.at[q]], add=True)
                return carry

            lax.fori_loop(0, nj, body, 0)
            plsc.subcore_barrier()

            for m in range(5):
                r0e = s * _RPT + m * _CH
                rn = _CH if m < 4 else _RPT - 4 * _CH
                pltpu.sync_copy(acc_sh.at[pl.ds(r0e, rn)],
                                rows_v.at[pl.ds(0, rn)])
                pltpu.sync_copy(rows_v.at[pl.ds(0, rn)],
                                outs[t].at[pl.ds(r0e, rn)])

            @pl.when(s == 0)
            def _tail():
                rt = _N - _NS * _RPT
                pltpu.sync_copy(acc_sh.at[pl.ds(_NS * _RPT, rt)],
                                rows_v.at[pl.ds(0, rt)])
                pltpu.sync_copy(rows_v.at[pl.ds(0, rt)],
                                outs[t].at[pl.ds(_NS * _RPT, rt)])

            plsc.subcore_barrier()


@functools.lru_cache(maxsize=1)
def _spmm_call():
    return pl.kernel(
        _spmm_body,
        out_type=[jax.ShapeDtypeStruct((_N, _D), _f32)] * _NT,
        mesh=plsc.VectorSubcoreMesh(core_axis_name="c", subcore_axis_name="s",
                                    num_cores=_NC, num_subcores=_NS),
        scratch_types=[
            pltpu.VMEM((4 * _CH,), jnp.int32),
            pltpu.VMEM((4, _CH), jnp.int32),
            pltpu.VMEM((2 * _CH, _D), _f32),
            pltpu.VMEM_SHARED((_PAD_N, _D), _f32),
            pltpu.SemaphoreType.DMA,
            pltpu.SemaphoreType.DMA,
        ],
    )


def _deg_body(dstp, zrow16, ones16,
              deg0, deg1, deg2, deg3,
              dst2d_v, ones_v, dstage_v, dacc_sh, dsem):
    c = lax.axis_index("c")
    s = lax.axis_index("s")
    degs = (deg0, deg1, deg2, deg3)

    pltpu.sync_copy(ones16, ones_v)
    pltpu.sync_copy(zrow16, dstage_v)

    for t in range(_NT):
        @pl.when(c == t // 2)
        def _process():
            r0 = t * (_EP // _CH) + s * _CPT
            pltpu.sync_copy(dstp.at[pl.ds(r0, _CPT)], dst2d_v)

            for m in range(5):
                rn = _CH if m < 4 else _ZR - 4 * _CH
                pltpu.sync_copy(dstage_v.at[pl.ds(0, rn)],
                                dacc_sh.at[pl.ds(s * _ZR + m * _CH, rn)])
            plsc.subcore_barrier()

            def fire(j, carry):
                pltpu.async_copy(ones_v, dacc_sh.at[dst2d_v.at[j]], dsem,
                                 add=True)
                return carry

            def drain(j, carry):
                pltpu.make_async_copy(
                    ones_v, dacc_sh.at[dst2d_v.at[j]], dsem).wait()
                return carry

            lax.fori_loop(0, _CPT, fire, 0)
            lax.fori_loop(0, _CPT, drain, 0)
            plsc.subcore_barrier()

            for m in range(5):
                r0e = s * _RPT + m * _CH
                rn = _CH if m < 4 else _RPT - 4 * _CH
                pltpu.sync_copy(dacc_sh.at[pl.ds(r0e, rn)],
                                dstage_v.at[pl.ds(0, rn)])
                pltpu.sync_copy(dstage_v.at[pl.ds(0, rn)],
                                degs[t].at[pl.ds(r0e, rn)])

            @pl.when(s == 0)
            def _tail():
                rt = _N - _NS * _RPT
                pltpu.sync_copy(dacc_sh.at[pl.ds(_NS * _RPT, rt)],
                                dstage_v.at[pl.ds(0, rt)])
                pltpu.sync_copy(dstage_v.at[pl.ds(0, rt)],
                                degs[t].at[pl.ds(_NS * _RPT, rt)])

            pltpu.sync_copy(zrow16, dstage_v)
            plsc.subcore_barrier()


@functools.lru_cache(maxsize=1)
def _deg_call():
    return pl.kernel(
        _deg_body,
        out_type=[jax.ShapeDtypeStruct((_N, _DW), _f32)] * _NT,
        mesh=plsc.VectorSubcoreMesh(core_axis_name="c", subcore_axis_name="s",
                                    num_cores=_NC, num_subcores=_NS),
        scratch_types=[
            pltpu.VMEM((_CPT, _CH), jnp.int32),
            pltpu.VMEM((_CH, _DW), _f32),
            pltpu.VMEM((_CH, _DW), _f32),
            pltpu.VMEM_SHARED((_PAD_N, _DW), _f32),
            pltpu.SemaphoreType.DMA,
        ],
    )


def kernel(node_feat, edge_index, w_n2l_W, w_n2l_b, conv_W, conv_b,
           merge_W, merge_b):
    src = edge_index[:, 0, :].reshape(-1)
    dst = edge_index[:, 1, :].reshape(-1)
    pad = _EP - _E
    pad_rows = _N + jnp.arange(pad, dtype=jnp.int32) % (_PAD_N - _N)
    dstp = jnp.concatenate(
        [edge_index[:, 1, :],
         jnp.broadcast_to(pad_rows, (_NT, pad))], axis=1).reshape(-1, _CH)
    zrow = jnp.zeros((_CH, _D), _f32)
    zrow16 = jnp.zeros((_CH, _DW), _f32)
    ones16 = jnp.ones((_CH, _DW), _f32)

    h = _embed(node_feat, w_n2l_W, w_n2l_b.reshape(1, _D))
    degs = _deg_call()(dstp, zrow16, ones16)
    for lv in range(_LV):
        gs = _spmm_call()(h, src, dst, zrow)
        h = _merge(gs, degs, h, conv_W[lv], conv_b[lv].reshape(1, _NT * _D),
                   merge_W[lv], merge_b[lv].reshape(1, _D))
    return h
